# Initial kernel scaffold; baseline (speedup 1.0000x reference)
import jax
import jax.numpy as jnp
from jax.experimental import pallas as pl


def kernel(src_features, src_labels, tgt_features):
    raise NotImplementedError("write your pallas kernel here")


# speedup vs baseline: 12.3720x; 12.3720x over previous
import functools

import jax
import jax.numpy as jnp
from jax import lax
from jax.experimental import pallas as pl
from jax.experimental.pallas import tpu as pltpu

N = 4096
D = 64
NUM_CLASSES = 64
K = 5
M = 5
MU = 2048

_CBLK = 256
_RBLK = 256
_SBLK = 512

_NEG_INF = float("-inf")
_BIG = 1 << 30


def _pass1_body(src_ref, tgtt_ref, lab_ref, ratio_ref, asg_ref):
    src = src_ref[...]
    tgtt = tgtt_ref[...]
    lab = lab_ref[...]
    c = tgtt.shape[1]

    a2 = jnp.sum(src * src, axis=1, keepdims=True)
    b2 = jnp.sum(tgtt * tgtt, axis=0, keepdims=True)
    ab = lax.dot_general(src, tgtt, (((1,), (0,)), ((), ())),
                         preferred_element_type=jnp.float32,
                         precision=lax.Precision.HIGHEST)
    d2 = jnp.maximum(a2 + b2 - 2.0 * ab, 1e-12)
    sim = 1.0 / (jnp.sqrt(d2) + 1.0)

    row = lax.broadcasted_iota(jnp.int32, (N, c), 0)
    keys = row * NUM_CLASSES + lab

    work = sim
    labs = []
    for _ in range(K):
        v = jnp.max(work, axis=0, keepdims=True)
        kk = jnp.min(jnp.where(work == v, keys, _BIG), axis=0, keepdims=True)
        labs.append(kk % NUM_CLASSES)
        idx = kk // NUM_CLASSES
        work = jnp.where(row == idx, _NEG_INF, work)

    score = []
    for i in range(K):
        cnt = labs[0] * 0
        for j in range(K):
            cnt = cnt + (labs[i] == labs[j]).astype(jnp.int32)
        score.append(cnt * (NUM_CLASSES + 1) + (NUM_CLASSES - labs[i]))
    smax = score[0]
    for i in range(1, K):
        smax = jnp.maximum(smax, score[i])
    assigned = labs[0] * 0 - 1
    for i in range(K):
        assigned = jnp.maximum(assigned, jnp.where(score[i] == smax, labs[i], -1))

    mask_same = lab == assigned

    def topm_sum(masked):
        acc = jnp.zeros((1, c), jnp.float32)
        w = masked
        for _ in range(M):
            v = jnp.max(w, axis=0, keepdims=True)
            acc = acc + v
            ik = jnp.min(jnp.where(w == v, row, _BIG), axis=0, keepdims=True)
            w = jnp.where(row == ik, _NEG_INF, w)
        return acc

    nln = topm_sum(jnp.where(mask_same, sim, _NEG_INF))
    nun = topm_sum(jnp.where(mask_same, _NEG_INF, sim))
    ratio = nln / nun
    ratio = jnp.where(jnp.isnan(ratio), _NEG_INF, ratio)

    ratio_ref[...] = jnp.broadcast_to(ratio, (8, c))
    asg_ref[...] = jnp.broadcast_to(assigned, (8, c))


def _rank_body(rrow_ref, rcol_ref, sel_ref):
    j = pl.program_id(0)
    a = rrow_ref[0:1, :]
    b = rcol_ref[...]
    bsz = a.shape[1]
    col = j * bsz + lax.broadcasted_iota(jnp.int32, (1, bsz), 1)
    rowi = lax.broadcasted_iota(jnp.int32, (N, bsz), 0)
    gt = (b > a).astype(jnp.float32)
    eqlt = ((b == a) & (rowi < col)).astype(jnp.float32)
    rank = jnp.sum(gt + eqlt, axis=0, keepdims=True)
    sel_ref[...] = jnp.broadcast_to((rank < MU).astype(jnp.float32), (8, bsz))


def _pass2_body(src_ref, tgtt_ref, lab_ref, sel_ref, asg_ref,
                loss_ref, cnt_ref):
    i = pl.program_id(0)
    src = src_ref[...]
    tgtt = tgtt_ref[...]
    lab = lab_ref[...]

    a2 = jnp.sum(src * src, axis=1, keepdims=True)
    b2 = jnp.sum(tgtt * tgtt, axis=0, keepdims=True)
    ab = lax.dot_general(src, tgtt, (((1,), (0,)), ((), ())),
                         preferred_element_type=jnp.float32,
                         precision=lax.Precision.HIGHEST)
    d2 = jnp.maximum(a2 + b2 - 2.0 * ab, 1e-12)
    sim = 1.0 / (jnp.sqrt(d2) + 1.0)

    sel = sel_ref[0:1, :]
    asg = asg_ref[0:1, :]
    match = (lab == asg).astype(jnp.float32)
    e = jnp.exp(sim) * sel
    den = jnp.sum(e, axis=1, keepdims=True)
    num = jnp.sum(e * match, axis=1, keepdims=True)
    same_cnt = jnp.sum(sel * match, axis=1, keepdims=True)
    diff_cnt = jnp.sum(sel * (1.0 - match), axis=1, keepdims=True)
    valid = (same_cnt > 0.0) & (diff_cnt > 0.0)
    term = jnp.where(valid, jnp.log(num / den), 0.0)

    @pl.when(i == 0)
    def _():
        loss_ref[0, 0] = jnp.float32(0.0)
        cnt_ref[0, 0] = jnp.float32(0.0)

    loss_ref[0, 0] += jnp.sum(term)
    cnt_ref[0, 0] += jnp.sum(valid.astype(jnp.float32))


@jax.jit
def kernel(src_features, src_labels, tgt_features):
    lab_col = src_labels.astype(jnp.int32).reshape(N, 1)
    tgt_t = tgt_features.T

    ratio8, asg8 = pl.pallas_call(
        _pass1_body,
        grid=(N // _CBLK,),
        in_specs=[
            pl.BlockSpec((N, D), lambda j: (0, 0)),
            pl.BlockSpec((D, _CBLK), lambda j: (0, j)),
            pl.BlockSpec((N, 1), lambda j: (0, 0)),
        ],
        out_specs=[
            pl.BlockSpec((8, _CBLK), lambda j: (0, j)),
            pl.BlockSpec((8, _CBLK), lambda j: (0, j)),
        ],
        out_shape=[
            jax.ShapeDtypeStruct((8, N), jnp.float32),
            jax.ShapeDtypeStruct((8, N), jnp.int32),
        ],
    )(src_features, tgt_t, lab_col)

    ratio_col = ratio8[0].reshape(N, 1)

    sel8 = pl.pallas_call(
        _rank_body,
        grid=(N // _SBLK,),
        in_specs=[
            pl.BlockSpec((8, _SBLK), lambda j: (0, j)),
            pl.BlockSpec((N, 1), lambda j: (0, 0)),
        ],
        out_specs=pl.BlockSpec((8, _SBLK), lambda j: (0, j)),
        out_shape=jax.ShapeDtypeStruct((8, N), jnp.float32),
    )(ratio8, ratio_col)

    loss_sum, n_valid = pl.pallas_call(
        _pass2_body,
        grid=(N // _RBLK,),
        in_specs=[
            pl.BlockSpec((_RBLK, D), lambda i: (i, 0)),
            pl.BlockSpec((D, N), lambda i: (0, 0)),
            pl.BlockSpec((_RBLK, 1), lambda i: (i, 0)),
            pl.BlockSpec((8, N), lambda i: (0, 0)),
            pl.BlockSpec((8, N), lambda i: (0, 0)),
        ],
        out_specs=[
            pl.BlockSpec(memory_space=pltpu.SMEM),
            pl.BlockSpec(memory_space=pltpu.SMEM),
        ],
        out_shape=[
            jax.ShapeDtypeStruct((1, 1), jnp.float32),
            jax.ShapeDtypeStruct((1, 1), jnp.float32),
        ],
    )(src_features, tgt_t, lab_col, sel8, asg8)

    return -loss_sum[0, 0] / jnp.maximum(n_valid[0, 0], 1.0)
